# Initial kernel scaffold; baseline (speedup 1.0000x reference)
import functools

import jax
import jax.numpy as jnp
from jax import lax
from jax.experimental import pallas as pl
from jax.experimental.pallas import tpu as pltpu
from jax.experimental.pallas import tpu_sc as plsc

B, T_H, T_Q, V, D = 4096, 200, 20, 100000, 64

_info = plsc.get_sparse_core_info()
NC, NS, L = _info.num_cores, _info.num_subcores, _info.num_lanes
NW = NC * NS
ROWS_W = B // NW
CH = 512
GSUB = 128
TOK_H = ROWS_W * T_H
TOK_Q = ROWS_W * T_Q


def _process_stream(wid, T, n_tok_w, tok_hbm, table, out_hbm, mask_hbm,
                    len_buf, tok_buf, rows_buf, mask_buf, keep_buf, sem):
  n_chunks = n_tok_w // CH
  base_w = wid * n_tok_w

  @pl.loop(0, n_chunks)
  def _chunk(g):
    tok_base = base_w + g * CH
    pltpu.sync_copy(tok_hbm.at[pl.ds(tok_base, CH)], tok_buf)

    descs = []
    for j in range(CH // GSUB):
      descs.append(pltpu.async_copy(
          table.at[tok_buf.at[pl.ds(j * GSUB, GSUB)]],
          rows_buf.at[pl.ds(j * GSUB, GSUB)], sem))

    local_base = g * CH
    iota = lax.iota(jnp.int32, L)
    for i in range(CH // L):
      pos = local_base + i * L + iota
      t = pos % T
      r = pos // T
      len_v = plsc.load_gather(len_buf, [r])
      tok_v = tok_buf[pl.ds(i * L, L)]
      m = t < len_v
      mask_buf[pl.ds(i * L, L)] = m.astype(jnp.int32)
      keep = jnp.logical_and(m, tok_v != 0)
      keep_buf[pl.ds(i * L, L)] = keep.astype(jnp.float32)

    for d in descs:
      d.wait()

    @pl.loop(0, CH)
    def _mul(k):
      ks = keep_buf[k]
      for q in range(D // L):
        rows_buf[k, pl.ds(q * L, L)] = rows_buf[k, pl.ds(q * L, L)] * ks

    pltpu.sync_copy(rows_buf, out_hbm.at[pl.ds(tok_base, CH)])
    pltpu.sync_copy(mask_buf, mask_hbm.at[pl.ds(tok_base, CH)])


def _sc_body(tok_h, tok_q, hlen, qlen, table,
             out_h, out_q, mask_h, mask_q,
             tok_buf, rows_buf, mask_buf, keep_buf, hlen_buf, qlen_buf, sem):
  wid = lax.axis_index("s") * NC + lax.axis_index("c")
  pltpu.sync_copy(hlen.at[pl.ds(wid * ROWS_W, ROWS_W)], hlen_buf)
  pltpu.sync_copy(qlen.at[pl.ds(wid * ROWS_W, ROWS_W)], qlen_buf)

  _process_stream(wid, T_H, TOK_H, tok_h, table, out_h, mask_h,
                  hlen_buf, tok_buf, rows_buf, mask_buf, keep_buf, sem)
  _process_stream(wid, T_Q, TOK_Q, tok_q, table, out_q, mask_q,
                  qlen_buf, tok_buf, rows_buf, mask_buf, keep_buf, sem)


@jax.jit
def _encode(ques_tokens, hist_tokens, ques_len, hist_len, table):
  mesh = plsc.VectorSubcoreMesh(core_axis_name="c", subcore_axis_name="s")
  kfn = pl.kernel(
      _sc_body,
      out_type=[
          jax.ShapeDtypeStruct((B * T_H, D), jnp.float32),
          jax.ShapeDtypeStruct((B * T_Q, D), jnp.float32),
          jax.ShapeDtypeStruct((B * T_H,), jnp.int32),
          jax.ShapeDtypeStruct((B * T_Q,), jnp.int32),
      ],
      mesh=mesh,
      scratch_types=[
          pltpu.VMEM((CH,), jnp.int32),
          pltpu.VMEM((CH, D), jnp.float32),
          pltpu.VMEM((CH,), jnp.int32),
          pltpu.VMEM((CH,), jnp.float32),
          pltpu.VMEM((ROWS_W,), jnp.int32),
          pltpu.VMEM((ROWS_W,), jnp.int32),
          pltpu.SemaphoreType.DMA,
      ],
  )
  out_h, out_q, mask_h, mask_q = kfn(
      hist_tokens.reshape(-1), ques_tokens.reshape(-1),
      hist_len, ques_len, table)
  return (out_h.reshape(B, T_H, D), out_q.reshape(B, T_Q, D),
          mask_h.reshape(B, T_H), mask_q.reshape(B, T_Q))


def kernel(ques_tokens, hist_tokens, ques_len, hist_len, text_embedding_weight):
  ques_tokens = ques_tokens.astype(jnp.int32)
  hist_tokens = hist_tokens.astype(jnp.int32)
  ques_len = ques_len.astype(jnp.int32)
  hist_len = hist_len.astype(jnp.int32)
  return _encode(ques_tokens, hist_tokens, ques_len, hist_len,
                 text_embedding_weight)


# speedup vs baseline: 3.2227x; 3.2227x over previous
import functools

import jax
import jax.numpy as jnp
from jax import lax
from jax.experimental import pallas as pl
from jax.experimental.pallas import tpu as pltpu
from jax.experimental.pallas import tpu_sc as plsc

B, T_H, T_Q, V, D = 4096, 200, 20, 100000, 64

_info = plsc.get_sparse_core_info()
NC, NS, L = _info.num_cores, _info.num_subcores, _info.num_lanes
NW = NC * NS
ROWS_W = B // NW
CH = 512
GSUB = 128
TOK_H = ROWS_W * T_H
TOK_Q = ROWS_W * T_Q


def _process_stream(wid, T, n_tok_w, tok_hbm, table, out_hbm, mask_hbm,
                    len_buf, tok_buf, rows_buf, mask_buf, keep_buf, sem):
  n_chunks = n_tok_w // CH
  base_w = wid * n_tok_w

  @pl.loop(0, n_chunks)
  def _chunk(g):
    tok_base = base_w + g * CH
    pltpu.sync_copy(tok_hbm.at[pl.ds(tok_base, CH)], tok_buf)

    descs = []
    for j in range(CH // GSUB):
      descs.append(pltpu.async_copy(
          table.at[tok_buf.at[pl.ds(j * GSUB, GSUB)]],
          rows_buf.at[pl.ds(j * GSUB, GSUB)], sem))

    base_v = lax.broadcast_in_dim(g * CH, (L,), ())
    t_v = jnp.full((L,), T, jnp.int32)
    zero_v = jnp.zeros((L,), jnp.int32)
    for i in range(CH // L):
      offs = jnp.arange(i * L, (i + 1) * L, dtype=jnp.int32)
      pos = base_v + offs
      r = lax.div(pos, t_v)
      t = pos - r * t_v
      len_v = plsc.load_gather(len_buf, [r])
      tok_v = tok_buf[pl.ds(i * L, L)]
      m = t < len_v
      mask_buf[pl.ds(i * L, L)] = m.astype(jnp.int32)
      keep = jnp.logical_and(m, tok_v != zero_v)
      keep_buf[pl.ds(i * L, L)] = keep.astype(jnp.float32)

    for d in descs:
      d.wait()

    @pl.loop(0, CH // L)
    def _mul(kb):
      kv = keep_buf[pl.ds(kb * L, L)]
      base = kb * L
      for lane in range(L):
        ksv = lax.broadcast_in_dim(kv[lane], (L,), ())
        for q in range(D // L):
          row = base + lane
          rows_buf[row, pl.ds(q * L, L)] = rows_buf[row, pl.ds(q * L, L)] * ksv

    pltpu.sync_copy(rows_buf, out_hbm.at[pl.ds(tok_base, CH)])
    pltpu.sync_copy(mask_buf, mask_hbm.at[pl.ds(tok_base, CH)])


def _sc_body(tok_h, tok_q, hlen, qlen, table,
             out_h, out_q, mask_h, mask_q,
             tok_buf, rows_buf, mask_buf, keep_buf, hlen_buf, qlen_buf, sem):
  wid = lax.axis_index("s") * NC + lax.axis_index("c")
  pltpu.sync_copy(hlen.at[pl.ds(wid * ROWS_W, ROWS_W)], hlen_buf)
  pltpu.sync_copy(qlen.at[pl.ds(wid * ROWS_W, ROWS_W)], qlen_buf)

  _process_stream(wid, T_H, TOK_H, tok_h, table, out_h, mask_h,
                  hlen_buf, tok_buf, rows_buf, mask_buf, keep_buf, sem)
  _process_stream(wid, T_Q, TOK_Q, tok_q, table, out_q, mask_q,
                  qlen_buf, tok_buf, rows_buf, mask_buf, keep_buf, sem)


@jax.jit
def _encode(ques_tokens, hist_tokens, ques_len, hist_len, table):
  mesh = plsc.VectorSubcoreMesh(core_axis_name="c", subcore_axis_name="s")
  kfn = pl.kernel(
      _sc_body,
      out_type=[
          jax.ShapeDtypeStruct((B * T_H, D), jnp.float32),
          jax.ShapeDtypeStruct((B * T_Q, D), jnp.float32),
          jax.ShapeDtypeStruct((B * T_H,), jnp.int32),
          jax.ShapeDtypeStruct((B * T_Q,), jnp.int32),
      ],
      mesh=mesh,
      compiler_params=pltpu.CompilerParams(
          use_tc_tiling_on_sc=False, needs_layout_passes=False),
      scratch_types=[
          pltpu.VMEM((CH,), jnp.int32),
          pltpu.VMEM((CH, D), jnp.float32),
          pltpu.VMEM((CH,), jnp.int32),
          pltpu.VMEM((CH,), jnp.float32),
          pltpu.VMEM((ROWS_W,), jnp.int32),
          pltpu.VMEM((ROWS_W,), jnp.int32),
          pltpu.SemaphoreType.DMA,
      ],
  )
  out_h, out_q, mask_h, mask_q = kfn(
      hist_tokens.reshape(-1), ques_tokens.reshape(-1),
      hist_len, ques_len, table)
  return (out_h.reshape(B, T_H, D), out_q.reshape(B, T_Q, D),
          mask_h.reshape(B, T_H), mask_q.reshape(B, T_Q))


def kernel(ques_tokens, hist_tokens, ques_len, hist_len, text_embedding_weight):
  ques_tokens = ques_tokens.astype(jnp.int32)
  hist_tokens = hist_tokens.astype(jnp.int32)
  ques_len = ques_len.astype(jnp.int32)
  hist_len = hist_len.astype(jnp.int32)
  return _encode(ques_tokens, hist_tokens, ques_len, hist_len,
                 text_embedding_weight)
